# baseline (speedup 1.0000x reference)
import jax
import jax.numpy as jnp
from jax.experimental import pallas as pl
from jax.experimental.pallas import tpu as pltpu

_TM = 80


def _matmul_kernel(adj_ref, w_ref, b_ref, out_ref):
    out_ref[...] = (
        jnp.dot(adj_ref[...], w_ref[...], preferred_element_type=jnp.float32)
        + b_ref[...]
    )


def kernel(adj, weight, bias):
    M, K = adj.shape
    _, N = weight.shape
    bias2 = bias.reshape(1, N)
    return pl.pallas_call(
        _matmul_kernel,
        grid=(M // _TM,),
        in_specs=[
            pl.BlockSpec((_TM, K), lambda i: (i, 0)),
            pl.BlockSpec((K, N), lambda i: (0, 0)),
            pl.BlockSpec((1, N), lambda i: (0, 0)),
        ],
        out_specs=pl.BlockSpec((_TM, N), lambda i: (i, 0)),
        out_shape=jax.ShapeDtypeStruct((M, N), jnp.float32),
        compiler_params=pltpu.CompilerParams(
            dimension_semantics=("parallel",),
        ),
    )(adj, weight, bias2)


# speedup vs baseline: 1.3339x; 1.3339x over previous
import jax
import jax.numpy as jnp
from jax.experimental import pallas as pl
from jax.experimental.pallas import tpu as pltpu

_TM = 200


def _matmul_kernel(adj_ref, w_ref, b_ref, out_ref):
    out_ref[...] = (
        jnp.dot(
            adj_ref[...].astype(jnp.bfloat16),
            w_ref[...].astype(jnp.bfloat16),
            preferred_element_type=jnp.float32,
        )
        + b_ref[...]
    )


def kernel(adj, weight, bias):
    M, K = adj.shape
    _, N = weight.shape
    bias2 = bias.reshape(1, N)
    return pl.pallas_call(
        _matmul_kernel,
        grid=(M // _TM,),
        in_specs=[
            pl.BlockSpec((_TM, K), lambda i: (i, 0)),
            pl.BlockSpec((K, N), lambda i: (0, 0)),
            pl.BlockSpec((1, N), lambda i: (0, 0)),
        ],
        out_specs=pl.BlockSpec((_TM, N), lambda i: (i, 0)),
        out_shape=jax.ShapeDtypeStruct((M, N), jnp.float32),
        compiler_params=pltpu.CompilerParams(
            dimension_semantics=("parallel",),
        ),
    )(adj, weight, bias2)


# speedup vs baseline: 1.3503x; 1.0123x over previous
import jax
import jax.numpy as jnp
from jax.experimental import pallas as pl
from jax.experimental.pallas import tpu as pltpu

_TM = 200


def _matmul_kernel(adj_ref, w_ref, b_ref, out_ref):
    out_ref[...] = (
        jnp.dot(adj_ref[...], w_ref[...], preferred_element_type=jnp.float32)
        + b_ref[...]
    )


def kernel(adj, weight, bias):
    M, K = adj.shape
    _, N = weight.shape
    bias2 = bias.reshape(1, N)
    return pl.pallas_call(
        _matmul_kernel,
        grid=(M // _TM,),
        in_specs=[
            pl.BlockSpec((_TM, K), lambda i: (i, 0)),
            pl.BlockSpec((K, N), lambda i: (0, 0)),
            pl.BlockSpec((1, N), lambda i: (0, 0)),
        ],
        out_specs=pl.BlockSpec((_TM, N), lambda i: (i, 0)),
        out_shape=jax.ShapeDtypeStruct((M, N), jnp.float32),
        compiler_params=pltpu.CompilerParams(
            dimension_semantics=("parallel",),
        ),
    )(adj, weight, bias2)
